# baseline (speedup 1.0000x reference)
import dataclasses
import functools

import jax
import jax.numpy as jnp
from jax import lax
from jax.experimental import pallas as pl
from jax.experimental.pallas import tpu as pltpu
from jax.experimental.pallas import tpu_sc as plsc

NUM_BUCKETS = 1000000
EMB = 128
LANES = 16
WINDOW = 128

_MULT_LO = 435761
_MULT_HI = 219264


def _hash16(v):
    xh = lax.shift_right_logical(v, jnp.int32(10))
    xl = lax.bitwise_and(v, jnp.int32(1023))
    s = xh * jnp.int32(_MULT_HI) + xl * jnp.int32(_MULT_LO)
    for k in (512, 256, 128, 64, 32, 16, 8, 4, 2, 1):
        t = jnp.int32(k * NUM_BUCKETS)
        s = jnp.where(s >= t, s - t, s)
    return s


_NWORKERS = 32
_NBUF = 4
_DEPTH = 3


def _sc_gather(xt, table):
    (n,) = xt.shape
    per_tile = n // _NWORKERS
    chunks = per_tile // WINDOW
    mesh = plsc.VectorSubcoreMesh(core_axis_name="core", subcore_axis_name="subcore")
    cp = pltpu.CompilerParams()
    if "needs_layout_passes" in pltpu.CompilerParams.__dataclass_fields__:
        cp = dataclasses.replace(cp, needs_layout_passes=False)

    @functools.partial(
        pl.kernel,
        out_type=jax.ShapeDtypeStruct((n, EMB), jnp.bfloat16),
        mesh=mesh,
        compiler_params=cp,
        scratch_types=[
            pltpu.VMEM((per_tile,), jnp.int32),
            pltpu.VMEM((per_tile,), jnp.int32),
            pltpu.VMEM((_NBUF, WINDOW, EMB), jnp.float32),
            pltpu.VMEM((_NBUF, WINDOW, EMB), jnp.bfloat16),
            pltpu.SemaphoreType.DMA,
        ]
        + [pltpu.SemaphoreType.DMA] * (2 * _NBUF),
    )
    def gather_kernel(
        x_hbm, table_hbm, o_hbm, ids_v, idx_v, gbuf_v, obuf_v, sem_in, *sems
    ):
        gsem = sems[:_NBUF]
        osem = sems[_NBUF:]
        core = lax.axis_index("core")
        sub = lax.axis_index("subcore")
        wid = sub * 2 + core
        base = pl.multiple_of(wid * per_tile, per_tile)

        pltpu.async_copy(x_hbm.at[pl.ds(base, per_tile)], ids_v, sem_in).wait()

        @pl.loop(0, per_tile, step=LANES * 8)
        def _(i):
            off = pl.multiple_of(i, LANES * 8)
            for j in range(8):
                v = ids_v[pl.ds(off + j * LANES, LANES)]
                idx_v[pl.ds(off + j * LANES, LANES)] = _hash16(v)

        def gather_copy(c):
            b = c % _NBUF
            return pltpu.make_async_copy(
                table_hbm.at[idx_v.at[pl.ds(c * WINDOW, WINDOW)]],
                gbuf_v.at[b],
                gsem[b],
            )

        def out_copy(c):
            b = c % _NBUF
            return pltpu.make_async_copy(
                obuf_v.at[b],
                o_hbm.at[pl.ds(base + c * WINDOW, WINDOW)],
                osem[b],
            )

        def convert(c):
            b = c % _NBUF
            src = gbuf_v.at[b]
            dst = obuf_v.at[b]

            @pl.loop(0, WINDOW)
            def _(r):
                for g in range(EMB // 32):
                    lo = src[r, pl.ds(g * 32, LANES)]
                    hi = src[r, pl.ds(g * 32 + LANES, LANES)]
                    dst[r, pl.ds(g * 32, 32)] = plsc.pack(
                        lo, hi, format=plsc.PackFormat.INTERLEAVED
                    )

        for c in range(min(_DEPTH, chunks)):
            gather_copy(c).start()
        for c in range(chunks):
            gather_copy(c).wait()
            if c >= _NBUF:
                out_copy(c - _NBUF).wait()
            convert(c)
            out_copy(c).start()
            k = c + _DEPTH
            if k < chunks:
                gather_copy(k).start()
        for c in range(max(chunks - _NBUF, 0), chunks):
            out_copy(c).wait()

    return gather_kernel(xt, table)


def _mlp(embeds_fm, w1t, b1, w2t, b2, bsz, nf):
    h = w1t.shape[1]
    o = w2t.shape[1]
    bm = 512
    nbm = bsz // bm

    def body(e_ref, w1_ref, b1_ref, w2_ref, b2_ref, o_ref):
        e_full = jnp.concatenate([e_ref[f] for f in range(nf)], axis=1)
        hid = jnp.dot(e_full, w1_ref[...], preferred_element_type=jnp.float32)
        hid = jnp.maximum(hid + b1_ref[...], 0.0)
        o_ref[...] = (
            jnp.dot(
                hid.astype(jnp.bfloat16),
                w2_ref[...],
                preferred_element_type=jnp.float32,
            )
            + b2_ref[...]
        )

    return pl.pallas_call(
        body,
        grid=(nbm,),
        in_specs=[
            pl.BlockSpec((nf, bm, EMB), lambda i: (0, i, 0)),
            pl.BlockSpec((nf * EMB, h), lambda i: (0, 0)),
            pl.BlockSpec((1, h), lambda i: (0, 0)),
            pl.BlockSpec((h, o), lambda i: (0, 0)),
            pl.BlockSpec((1, o), lambda i: (0, 0)),
        ],
        out_specs=pl.BlockSpec((bm, o), lambda i: (i, 0)),
        out_shape=jax.ShapeDtypeStruct((bsz, o), jnp.float32),
    )(embeds_fm, w1t, b1, w2t, b2)


_NCHUNKS = 1


def _permute_w1_rows(w1t, nf):
    h = w1t.shape[1]
    w = w1t.reshape(nf, EMB // 32, 2, 16, h)
    return w.transpose(0, 1, 3, 2, 4).reshape(nf * EMB, h)


def kernel(x, table, W1, b1, W2, b2):
    bsz, nf = x.shape
    cb = bsz // _NCHUNKS
    xt = x.astype(jnp.int32).T
    w1t = _permute_w1_rows(W1.T.astype(jnp.bfloat16), nf)
    w2t = W2.T.astype(jnp.bfloat16)
    b1r = b1.reshape(1, -1)
    b2r = b2.reshape(1, -1)
    with jax.enable_x64(False):
        outs = []
        for h in range(_NCHUNKS):
            xth = lax.slice(xt, (0, h * cb), (nf, (h + 1) * cb)).reshape(-1)
            embeds_fm = _sc_gather(xth, table)
            outs.append(
                _mlp(embeds_fm.reshape(nf, cb, EMB), w1t, b1r, w2t, b2r, cb, nf)
            )
        out = jnp.concatenate(outs, axis=0) if _NCHUNKS > 1 else outs[0]
    return out


# speedup vs baseline: 1.2801x; 1.2801x over previous
import dataclasses
import functools

import jax
import jax.numpy as jnp
from jax import lax
from jax.experimental import pallas as pl
from jax.experimental.pallas import tpu as pltpu
from jax.experimental.pallas import tpu_sc as plsc

NUM_BUCKETS = 1000000
EMB = 128
LANES = 16
WINDOW = 128

_MULT_LO = 435761
_MULT_HI = 219264


def _hash16(v):
    xh = lax.shift_right_logical(v, jnp.int32(10))
    xl = lax.bitwise_and(v, jnp.int32(1023))
    s = xh * jnp.int32(_MULT_HI) + xl * jnp.int32(_MULT_LO)
    for k in (512, 256, 128, 64, 32, 16, 8, 4, 2, 1):
        t = jnp.int32(k * NUM_BUCKETS)
        s = jnp.where(s >= t, s - t, s)
    return s


_NWORKERS = 32
_NBUF = 6
_DEPTH = 3


def _sc_gather(xt, table):
    (n,) = xt.shape
    per_tile = n // _NWORKERS
    chunks = per_tile // WINDOW
    mesh = plsc.VectorSubcoreMesh(core_axis_name="core", subcore_axis_name="subcore")

    @functools.partial(
        pl.kernel,
        out_type=jax.ShapeDtypeStruct((n, EMB), jnp.float32),
        mesh=mesh,
        scratch_types=[
            pltpu.VMEM((per_tile,), jnp.int32),
            pltpu.VMEM((per_tile,), jnp.int32),
            pltpu.VMEM((_NBUF, WINDOW, EMB), jnp.float32),
            pltpu.SemaphoreType.DMA,
        ]
        + [pltpu.SemaphoreType.DMA] * (2 * _NBUF),
    )
    def gather_kernel(x_hbm, table_hbm, o_hbm, ids_v, idx_v, gbuf_v, sem_in, *sems):
        gsem = sems[:_NBUF]
        osem = sems[_NBUF:]
        core = lax.axis_index("core")
        sub = lax.axis_index("subcore")
        wid = sub * 2 + core
        base = pl.multiple_of(wid * per_tile, per_tile)

        pltpu.async_copy(x_hbm.at[pl.ds(base, per_tile)], ids_v, sem_in).wait()

        @pl.loop(0, per_tile, step=LANES * 8)
        def _(i):
            off = pl.multiple_of(i, LANES * 8)
            for j in range(8):
                v = ids_v[pl.ds(off + j * LANES, LANES)]
                idx_v[pl.ds(off + j * LANES, LANES)] = _hash16(v)

        def gather_copy(c):
            b = c % _NBUF
            return pltpu.make_async_copy(
                table_hbm.at[idx_v.at[pl.ds(c * WINDOW, WINDOW)]],
                gbuf_v.at[b],
                gsem[b],
            )

        def out_copy(c):
            b = c % _NBUF
            return pltpu.make_async_copy(
                gbuf_v.at[b],
                o_hbm.at[pl.ds(base + c * WINDOW, WINDOW)],
                osem[b],
            )

        for c in range(min(_DEPTH, chunks)):
            gather_copy(c).start()
        for c in range(chunks):
            gather_copy(c).wait()
            out_copy(c).start()
            k = c + _DEPTH
            if k < chunks:
                if k >= _NBUF:
                    out_copy(k - _NBUF).wait()
                gather_copy(k).start()
        for c in range(max(chunks - _NBUF, 0), chunks):
            out_copy(c).wait()

    return gather_kernel(xt, table)


def _mlp(embeds_fm, w1t, b1, w2t, b2, bsz, nf):
    h = w1t.shape[1]
    o = w2t.shape[1]
    bm = 512
    nbm = bsz // bm

    def body(e_ref, w1_ref, b1_ref, w2_ref, b2_ref, o_ref):
        e_full = jnp.concatenate(
            [e_ref[f].astype(jnp.bfloat16) for f in range(nf)], axis=1
        )
        hid = jnp.dot(e_full, w1_ref[...], preferred_element_type=jnp.float32)
        hid = jnp.maximum(hid + b1_ref[...], 0.0)
        o_ref[...] = (
            jnp.dot(
                hid.astype(jnp.bfloat16),
                w2_ref[...],
                preferred_element_type=jnp.float32,
            )
            + b2_ref[...]
        )

    return pl.pallas_call(
        body,
        grid=(nbm,),
        in_specs=[
            pl.BlockSpec((nf, bm, EMB), lambda i: (0, i, 0)),
            pl.BlockSpec((nf * EMB, h), lambda i: (0, 0)),
            pl.BlockSpec((1, h), lambda i: (0, 0)),
            pl.BlockSpec((h, o), lambda i: (0, 0)),
            pl.BlockSpec((1, o), lambda i: (0, 0)),
        ],
        out_specs=pl.BlockSpec((bm, o), lambda i: (i, 0)),
        out_shape=jax.ShapeDtypeStruct((bsz, o), jnp.float32),
    )(embeds_fm, w1t, b1, w2t, b2)


_NCHUNKS = 1


def kernel(x, table, W1, b1, W2, b2):
    bsz, nf = x.shape
    cb = bsz // _NCHUNKS
    xt = x.astype(jnp.int32).T
    w1t = W1.T.astype(jnp.bfloat16)
    w2t = W2.T.astype(jnp.bfloat16)
    b1r = b1.reshape(1, -1)
    b2r = b2.reshape(1, -1)
    with jax.enable_x64(False):
        outs = []
        for h in range(_NCHUNKS):
            xth = lax.slice(xt, (0, h * cb), (nf, (h + 1) * cb)).reshape(-1)
            embeds_fm = _sc_gather(xth, table)
            outs.append(
                _mlp(embeds_fm.reshape(nf, cb, EMB), w1t, b1r, w2t, b2r, cb, nf)
            )
        out = jnp.concatenate(outs, axis=0) if _NCHUNKS > 1 else outs[0]
    return out


# speedup vs baseline: 1.2840x; 1.0030x over previous
import dataclasses
import functools

import jax
import jax.numpy as jnp
from jax import lax
from jax.experimental import pallas as pl
from jax.experimental.pallas import tpu as pltpu
from jax.experimental.pallas import tpu_sc as plsc

NUM_BUCKETS = 1000000
EMB = 128
LANES = 16
WINDOW = 128

_MULT_LO = 435761
_MULT_HI = 219264


def _hash16(v):
    xh = lax.shift_right_logical(v, jnp.int32(10))
    xl = lax.bitwise_and(v, jnp.int32(1023))
    s = xh * jnp.int32(_MULT_HI) + xl * jnp.int32(_MULT_LO)
    for k in (512, 256, 128, 64, 32, 16, 8, 4, 2, 1):
        t = jnp.int32(k * NUM_BUCKETS)
        s = jnp.where(s >= t, s - t, s)
    return s


_NWORKERS = 32
_NBUF = 6
_DEPTH = 4


def _sc_gather(xt, table):
    (n,) = xt.shape
    per_tile = n // _NWORKERS
    chunks = per_tile // WINDOW
    mesh = plsc.VectorSubcoreMesh(core_axis_name="core", subcore_axis_name="subcore")

    @functools.partial(
        pl.kernel,
        out_type=jax.ShapeDtypeStruct((n, EMB), jnp.float32),
        mesh=mesh,
        scratch_types=[
            pltpu.VMEM((per_tile,), jnp.int32),
            pltpu.VMEM((per_tile,), jnp.int32),
            pltpu.VMEM((_NBUF, WINDOW, EMB), jnp.float32),
            pltpu.SemaphoreType.DMA,
        ]
        + [pltpu.SemaphoreType.DMA] * (2 * _NBUF),
    )
    def gather_kernel(x_hbm, table_hbm, o_hbm, ids_v, idx_v, gbuf_v, sem_in, *sems):
        gsem = sems[:_NBUF]
        osem = sems[_NBUF:]
        core = lax.axis_index("core")
        sub = lax.axis_index("subcore")
        wid = sub * 2 + core
        base = pl.multiple_of(wid * per_tile, per_tile)

        pltpu.async_copy(x_hbm.at[pl.ds(base, per_tile)], ids_v, sem_in).wait()

        @pl.loop(0, per_tile, step=LANES * 8)
        def _(i):
            off = pl.multiple_of(i, LANES * 8)
            for j in range(8):
                v = ids_v[pl.ds(off + j * LANES, LANES)]
                idx_v[pl.ds(off + j * LANES, LANES)] = _hash16(v)

        def gather_copy(c):
            b = c % _NBUF
            return pltpu.make_async_copy(
                table_hbm.at[idx_v.at[pl.ds(c * WINDOW, WINDOW)]],
                gbuf_v.at[b],
                gsem[b],
            )

        def out_copy(c):
            b = c % _NBUF
            return pltpu.make_async_copy(
                gbuf_v.at[b],
                o_hbm.at[pl.ds(base + c * WINDOW, WINDOW)],
                osem[b],
            )

        for c in range(min(_DEPTH, chunks)):
            gather_copy(c).start()
        for c in range(chunks):
            gather_copy(c).wait()
            out_copy(c).start()
            k = c + _DEPTH
            if k < chunks:
                if k >= _NBUF:
                    out_copy(k - _NBUF).wait()
                gather_copy(k).start()
        for c in range(max(chunks - _NBUF, 0), chunks):
            out_copy(c).wait()

    return gather_kernel(xt, table)


def _mlp(embeds_fm, w1t, b1, w2t, b2, bsz, nf):
    h = w1t.shape[1]
    o = w2t.shape[1]
    bm = 512
    nbm = bsz // bm

    def body(e_ref, w1_ref, b1_ref, w2_ref, b2_ref, o_ref):
        e_full = jnp.concatenate(
            [e_ref[f].astype(jnp.bfloat16) for f in range(nf)], axis=1
        )
        hid = jnp.dot(e_full, w1_ref[...], preferred_element_type=jnp.float32)
        hid = jnp.maximum(hid + b1_ref[...], 0.0)
        o_ref[...] = (
            jnp.dot(
                hid.astype(jnp.bfloat16),
                w2_ref[...],
                preferred_element_type=jnp.float32,
            )
            + b2_ref[...]
        )

    return pl.pallas_call(
        body,
        grid=(nbm,),
        in_specs=[
            pl.BlockSpec((nf, bm, EMB), lambda i: (0, i, 0)),
            pl.BlockSpec((nf * EMB, h), lambda i: (0, 0)),
            pl.BlockSpec((1, h), lambda i: (0, 0)),
            pl.BlockSpec((h, o), lambda i: (0, 0)),
            pl.BlockSpec((1, o), lambda i: (0, 0)),
        ],
        out_specs=pl.BlockSpec((bm, o), lambda i: (i, 0)),
        out_shape=jax.ShapeDtypeStruct((bsz, o), jnp.float32),
    )(embeds_fm, w1t, b1, w2t, b2)


_NCHUNKS = 1


def kernel(x, table, W1, b1, W2, b2):
    bsz, nf = x.shape
    cb = bsz // _NCHUNKS
    xt = x.astype(jnp.int32).T
    w1t = W1.T.astype(jnp.bfloat16)
    w2t = W2.T.astype(jnp.bfloat16)
    b1r = b1.reshape(1, -1)
    b2r = b2.reshape(1, -1)
    with jax.enable_x64(False):
        outs = []
        for h in range(_NCHUNKS):
            xth = lax.slice(xt, (0, h * cb), (nf, (h + 1) * cb)).reshape(-1)
            embeds_fm = _sc_gather(xth, table)
            outs.append(
                _mlp(embeds_fm.reshape(nf, cb, EMB), w1t, b1r, w2t, b2r, cb, nf)
            )
        out = jnp.concatenate(outs, axis=0) if _NCHUNKS > 1 else outs[0]
    return out


# speedup vs baseline: 1.2964x; 1.0096x over previous
import dataclasses
import functools

import jax
import jax.numpy as jnp
from jax import lax
from jax.experimental import pallas as pl
from jax.experimental.pallas import tpu as pltpu
from jax.experimental.pallas import tpu_sc as plsc

NUM_BUCKETS = 1000000
EMB = 128
LANES = 16
WINDOW = 128

_MULT_LO = 435761
_MULT_HI = 219264


def _hash16(v):
    xh = lax.shift_right_logical(v, jnp.int32(10))
    xl = lax.bitwise_and(v, jnp.int32(1023))
    s = xh * jnp.int32(_MULT_HI) + xl * jnp.int32(_MULT_LO)
    for k in (512, 256, 128, 64, 32, 16, 8, 4, 2, 1):
        t = jnp.int32(k * NUM_BUCKETS)
        s = jnp.where(s >= t, s - t, s)
    return s


_NWORKERS = 32
_NBUF = 6
_DEPTH = 4


def _sc_gather(xt, table):
    (n,) = xt.shape
    per_tile = n // _NWORKERS
    chunks = per_tile // WINDOW
    mesh = plsc.VectorSubcoreMesh(core_axis_name="core", subcore_axis_name="subcore")

    @functools.partial(
        pl.kernel,
        out_type=jax.ShapeDtypeStruct((n, EMB), jnp.float32),
        mesh=mesh,
        scratch_types=[
            pltpu.VMEM((per_tile,), jnp.int32),
            pltpu.VMEM((per_tile,), jnp.int32),
            pltpu.VMEM((_NBUF, WINDOW, EMB), jnp.float32),
            pltpu.SemaphoreType.DMA,
        ]
        + [pltpu.SemaphoreType.DMA] * (2 * _NBUF),
    )
    def gather_kernel(x_hbm, table_hbm, o_hbm, ids_v, idx_v, gbuf_v, sem_in, *sems):
        gsem = sems[:_NBUF]
        osem = sems[_NBUF:]
        core = lax.axis_index("core")
        sub = lax.axis_index("subcore")
        wid = sub * 2 + core
        base = pl.multiple_of(wid * per_tile, per_tile)

        pltpu.async_copy(x_hbm.at[pl.ds(base, per_tile)], ids_v, sem_in).wait()

        head = min(_DEPTH * WINDOW, per_tile)

        @pl.loop(0, head, step=LANES * 8)
        def _(i):
            off = pl.multiple_of(i, LANES * 8)
            for j in range(8):
                v = ids_v[pl.ds(off + j * LANES, LANES)]
                idx_v[pl.ds(off + j * LANES, LANES)] = _hash16(v)

        def gather_copy(c):
            b = c % _NBUF
            return pltpu.make_async_copy(
                table_hbm.at[idx_v.at[pl.ds(c * WINDOW, WINDOW)]],
                gbuf_v.at[b],
                gsem[b],
            )

        def out_copy(c):
            b = c % _NBUF
            return pltpu.make_async_copy(
                gbuf_v.at[b],
                o_hbm.at[pl.ds(base + c * WINDOW, WINDOW)],
                osem[b],
            )

        for c in range(min(_DEPTH, chunks)):
            gather_copy(c).start()

        @pl.loop(head, per_tile, step=LANES * 8)
        def _(i):
            off = pl.multiple_of(i, LANES * 8)
            for j in range(8):
                v = ids_v[pl.ds(off + j * LANES, LANES)]
                idx_v[pl.ds(off + j * LANES, LANES)] = _hash16(v)

        for c in range(chunks):
            gather_copy(c).wait()
            out_copy(c).start()
            k = c + _DEPTH
            if k < chunks:
                if k >= _NBUF:
                    out_copy(k - _NBUF).wait()
                gather_copy(k).start()
        for c in range(max(chunks - _NBUF, 0), chunks):
            out_copy(c).wait()

    return gather_kernel(xt, table)


def _mlp(embeds_fm, w1t, b1, w2t, b2, bsz, nf):
    h = w1t.shape[1]
    o = w2t.shape[1]
    bm = 512
    nbm = bsz // bm

    def body(e_ref, w1_ref, b1_ref, w2_ref, b2_ref, o_ref):
        e_full = jnp.concatenate(
            [e_ref[f].astype(jnp.bfloat16) for f in range(nf)], axis=1
        )
        hid = jnp.dot(e_full, w1_ref[...], preferred_element_type=jnp.float32)
        hid = jnp.maximum(hid + b1_ref[...], 0.0)
        o_ref[...] = (
            jnp.dot(
                hid.astype(jnp.bfloat16),
                w2_ref[...],
                preferred_element_type=jnp.float32,
            )
            + b2_ref[...]
        )

    return pl.pallas_call(
        body,
        grid=(nbm,),
        in_specs=[
            pl.BlockSpec((nf, bm, EMB), lambda i: (0, i, 0)),
            pl.BlockSpec((nf * EMB, h), lambda i: (0, 0)),
            pl.BlockSpec((1, h), lambda i: (0, 0)),
            pl.BlockSpec((h, o), lambda i: (0, 0)),
            pl.BlockSpec((1, o), lambda i: (0, 0)),
        ],
        out_specs=pl.BlockSpec((bm, o), lambda i: (i, 0)),
        out_shape=jax.ShapeDtypeStruct((bsz, o), jnp.float32),
    )(embeds_fm, w1t, b1, w2t, b2)


_NCHUNKS = 1


def kernel(x, table, W1, b1, W2, b2):
    bsz, nf = x.shape
    cb = bsz // _NCHUNKS
    xt = x.astype(jnp.int32).T
    w1t = W1.T.astype(jnp.bfloat16)
    w2t = W2.T.astype(jnp.bfloat16)
    b1r = b1.reshape(1, -1)
    b2r = b2.reshape(1, -1)
    with jax.enable_x64(False):
        outs = []
        for h in range(_NCHUNKS):
            xth = lax.slice(xt, (0, h * cb), (nf, (h + 1) * cb)).reshape(-1)
            embeds_fm = _sc_gather(xth, table)
            outs.append(
                _mlp(embeds_fm.reshape(nf, cb, EMB), w1t, b1r, w2t, b2r, cb, nf)
            )
        out = jnp.concatenate(outs, axis=0) if _NCHUNKS > 1 else outs[0]
    return out
